# Initial kernel scaffold; baseline (speedup 1.0000x reference)
import jax
import jax.numpy as jnp
from jax.experimental import pallas as pl


def kernel(bert_0, bert_1, bert_2, W_self_0, W_neigh_0, b_0, W_self_1, W_neigh_1, b_1):
    raise NotImplementedError("write your pallas kernel here")


# speedup vs baseline: 2.1512x; 2.1512x over previous
import functools

import jax
import jax.numpy as jnp
from jax import lax
from jax.experimental import pallas as pl
from jax.experimental.pallas import tpu as pltpu
from jax.experimental.pallas import tpu_sc as plsc

_B = 1024
_F0 = 25
_F1 = 10
_D = 128

_NC = 2
_NS = 16
_NW = _NC * _NS

_G = _B * _F0
_GPW = _G // _NW
_CH = 40
_NCHUNK = _GPW // _CH


def _sc_segmean_body(x_hbm, out_hbm, inbuf, outbuf, sem_in, sem_out):
    wid = lax.axis_index("s") * _NC + lax.axis_index("c")
    in_base = wid * (_GPW * _F1)
    out_base = wid * _GPW

    def in_copy(g, slot):
        return pltpu.make_async_copy(
            x_hbm.at[pl.ds(in_base + g * (_CH * _F1), _CH * _F1)],
            inbuf.at[slot], sem_in)

    def out_copy(g, slot):
        return pltpu.make_async_copy(
            outbuf.at[slot],
            out_hbm.at[pl.ds(out_base + g * _CH, _CH)], sem_out)

    in_copy(0, 0).start()
    in_copy(1, 1).start()

    def chunk(g, slot):
        in_copy(g, slot).wait()

        def row(i, carry):
            for c in range(_D // 16):
                sl = pl.ds(c * 16, 16)
                acc = inbuf[slot, i * _F1, sl]
                for r in range(1, _F1):
                    acc = acc + inbuf[slot, i * _F1 + r, sl]
                outbuf[slot, i, sl] = acc * (1.0 / _F1)
            return carry

        lax.fori_loop(0, _CH, row, 0)
        out_copy(g, slot).start()

    def outer(t, carry):
        for b in range(2):
            g = t * 2 + b

            @pl.when(g >= 2)
            def _wait_out():
                out_copy(g - 2, b).wait()

            chunk(g, b)

            @pl.when(g + 2 < _NCHUNK)
            def _next_in():
                in_copy(g + 2, b).start()

        return carry

    lax.fori_loop(0, _NCHUNK // 2, outer, 0)
    out_copy(_NCHUNK - 2, 0).wait()
    out_copy(_NCHUNK - 1, 1).wait()


@functools.cache
def _sc_segmean():
    return pl.kernel(
        _sc_segmean_body,
        out_type=jax.ShapeDtypeStruct((_G, _D), jnp.float32),
        mesh=plsc.VectorSubcoreMesh(core_axis_name="c", subcore_axis_name="s",
                                    num_cores=_NC, num_subcores=_NS),
        scratch_types=[
            pltpu.VMEM((2, _CH * _F1, _D), jnp.float32),
            pltpu.VMEM((2, _CH, _D), jnp.float32),
            pltpu.SemaphoreType.DMA,
            pltpu.SemaphoreType.DMA,
        ],
    )


_NBLK = 8
_RPB = _G // _NBLK
_GPB = _B // _NBLK


def _tc_fused_body(b0_ref, x1_ref, m2_ref, ws0_ref, wn0_ref, bias0_ref,
                   ws1_ref, wn1_ref, bias1_ref, out_ref, m1h_ref, m1_ref):
    k = pl.program_id(0)
    x1 = x1_ref[...]
    act = jnp.maximum(
        jnp.dot(x1, ws0_ref[...], preferred_element_type=jnp.float32)
        + jnp.dot(m2_ref[...], wn0_ref[...], preferred_element_type=jnp.float32)
        + bias0_ref[...], 0.0)
    m1h_ref[pl.ds(k * _GPB, _GPB), :] = jnp.mean(
        act.reshape(_GPB, _F0, _D), axis=1)
    m1_ref[pl.ds(k * _GPB, _GPB), :] = jnp.mean(
        x1.reshape(_GPB, _F0, _D), axis=1)

    @pl.when(k == _NBLK - 1)
    def _finish():
        h0 = jnp.maximum(
            jnp.dot(b0_ref[...], ws0_ref[...], preferred_element_type=jnp.float32)
            + jnp.dot(m1_ref[...], wn0_ref[...], preferred_element_type=jnp.float32)
            + bias0_ref[...], 0.0)
        out_ref[...] = (
            jnp.dot(h0, ws1_ref[...], preferred_element_type=jnp.float32)
            + jnp.dot(m1h_ref[...], wn1_ref[...], preferred_element_type=jnp.float32)
            + bias1_ref[...])


def _tc_fused(bert_0, bert_1, m2, ws0, wn0, b0, ws1, wn1, b1):
    wspec = pl.BlockSpec((_D, _D), lambda k: (0, 0))
    bspec = pl.BlockSpec((1, _D), lambda k: (0, 0))
    return pl.pallas_call(
        _tc_fused_body,
        grid=(_NBLK,),
        in_specs=[
            pl.BlockSpec((_B, _D), lambda k: (0, 0)),
            pl.BlockSpec((_RPB, _D), lambda k: (k, 0)),
            pl.BlockSpec((_RPB, _D), lambda k: (k, 0)),
            wspec, wspec, bspec, wspec, wspec, bspec,
        ],
        out_specs=pl.BlockSpec((_B, _D), lambda k: (0, 0)),
        out_shape=jax.ShapeDtypeStruct((_B, _D), jnp.float32),
        scratch_shapes=[
            pltpu.VMEM((_B, _D), jnp.float32),
            pltpu.VMEM((_B, _D), jnp.float32),
        ],
    )(bert_0, bert_1, m2, ws0, wn0, b0.reshape(1, _D), ws1, wn1,
      b1.reshape(1, _D))


def kernel(bert_0, bert_1, bert_2, W_self_0, W_neigh_0, b_0,
           W_self_1, W_neigh_1, b_1):
    m2 = _sc_segmean()(bert_2)
    return _tc_fused(bert_0, bert_1, m2, W_self_0, W_neigh_0, b_0,
                     W_self_1, W_neigh_1, b_1)
